# baseline (speedup 1.0000x reference)
import functools

import jax
import jax.numpy as jnp
from jax import lax
from jax.experimental import pallas as pl
from jax.experimental.pallas import tpu as pltpu
from jax.experimental.pallas import tpu_sc as plsc

B, T, D, LT, NQ = 32, 1024, 1024, 32, 32
TR = 512


def _dot(a, b):
    bf = jnp.bfloat16
    K = a.shape[-1]
    acc = None
    for k0 in range(0, K, 256):
        d = jnp.dot(a[:, k0:k0 + 256].astype(bf), b[k0:k0 + 256, :].astype(bf),
                    preferred_element_type=jnp.float32)
        acc = d if acc is None else acc + d
    return acc


def _txt_pool_body(txt_ref, w_ref, b_ref, tm_ref, out_ref):
    x = txt_ref[0]
    h = jnp.tanh(_dot(x, w_ref[...]) + b_ref[...]) * tm_ref[0]
    s = jnp.sum(h, axis=0, keepdims=True)
    cnt = jnp.sum(tm_ref[0])
    out_ref[0] = s / jnp.maximum(cnt, 1.0)


def _txt_pool(txt_feat, W_txt, b_txt, tm3):
    return pl.pallas_call(
        _txt_pool_body,
        grid=(B,),
        in_specs=[
            pl.BlockSpec((1, LT, D), lambda b: (b, 0, 0)),
            pl.BlockSpec((D, D), lambda b: (0, 0)),
            pl.BlockSpec((1, D), lambda b: (0, 0)),
            pl.BlockSpec((1, LT, 1), lambda b: (b, 0, 0)),
        ],
        out_specs=pl.BlockSpec((1, 1, D), lambda b: (b, 0, 0)),
        out_shape=jax.ShapeDtypeStruct((B, 1, D), jnp.float32),
    )(txt_feat, W_txt, b_txt, tm3)


def _logit_body(a_ref, wb_ref, bb_ref, out_ref):
    lg = _dot(a_ref[0], wb_ref[...])
    out_ref[0, 0, :] = lg[:, 0] + bb_ref[0, 0]


def _logit_proj(amid, wb_pad, bb2):
    return pl.pallas_call(
        _logit_body,
        grid=(B, T // TR),
        in_specs=[
            pl.BlockSpec((1, TR, D), lambda b, t: (b, t, 0)),
            pl.BlockSpec((D, 128), lambda b, t: (0, 0)),
            pl.BlockSpec((1, 1), lambda b, t: (0, 0)),
        ],
        out_specs=pl.BlockSpec((1, 1, TR), lambda b, t: (b, 0, t)),
        out_shape=jax.ShapeDtypeStruct((B, 1, T), jnp.float32),
    )(amid, wb_pad, bb2)


def _sc_topk_gather(lg2, vid2):
    info = plsc.get_sparse_core_info()
    NC = info.num_cores
    mesh = plsc.VectorSubcoreMesh(core_axis_name="c", subcore_axis_name="s")

    @functools.partial(
        pl.kernel,
        mesh=mesh,
        out_type=[jax.ShapeDtypeStruct((B, NQ), jnp.float32),
                  jax.ShapeDtypeStruct((B * NQ, D), jnp.float32)],
        scratch_types=[pltpu.VMEM((T,), jnp.float32),
                       pltpu.VMEM((NQ,), jnp.float32),
                       pltpu.VMEM((NQ,), jnp.int32),
                       pltpu.VMEM((NQ, D), jnp.float32),
                       pltpu.SemaphoreType.DMA],
    )
    def sc_k(lg_hbm, vid_hbm, rc_hbm, g_hbm, lg_v, rc_v, ig_v, rows_v, sem):
        b = lax.axis_index("s") * NC + lax.axis_index("c")
        pltpu.sync_copy(lg_hbm.at[b], lg_v)
        lanes = lax.iota(jnp.int32, 16)
        neg = jnp.full((16,), -3.0e38, jnp.float32)
        zero_i = jnp.zeros((16,), jnp.int32)

        big = jnp.full((16,), 2 ** 30, jnp.int32)
        zerof = jnp.zeros((16,), jnp.float32)
        lane0 = lanes == jnp.zeros((16,), jnp.int32)

        def _butterfly(x, op):
            for s in (8, 4, 2, 1):
                idx = jnp.bitwise_xor(lanes, jnp.full((16,), s, jnp.int32))
                x = op(x, x.at[idx].get(mode="promise_in_bounds"))
            return x

        def extract(j, carry):
            rc_lo, rc_hi, ig_lo, ig_hi, gi_prev = carry

            def scan_chunk(c, mc):
                m, mi = mc
                v = lg_v[pl.ds(c * 16, 16)]
                idxv = lanes + jnp.full((16,), c * 16, jnp.int32)
                v = jnp.where(idxv == gi_prev, neg, v)
                lg_v[pl.ds(c * 16, 16)] = v
                gt = v > m
                return (jnp.where(gt, v, m), jnp.where(gt, idxv, mi))

            m, mi = lax.fori_loop(0, T // 16, scan_chunk, (neg, zero_i))
            gm_vec = _butterfly(m, jnp.maximum)
            sel = jnp.where(m == gm_vec, mi, big)
            gi_vec = _butterfly(sel, jnp.minimum)
            rc_vec = gi_vec.astype(jnp.float32) * jnp.full(
                (16,), 1.0 / T, jnp.float32)
            ij_vec = (rc_vec * jnp.full((16,), float(T - 1), jnp.float32)
                      + jnp.full((16,), 0.5, jnp.float32)).astype(jnp.int32)
            grow = jnp.full((16,), b * T, jnp.int32) + ij_vec
            lane_lo = jnp.where(j < 16, j, 999)
            lane_hi = jnp.where(j >= 16, j - 16, 999)
            m_lo = lanes == jnp.full((16,), lane_lo, jnp.int32)
            m_hi = lanes == jnp.full((16,), lane_hi, jnp.int32)
            rc_lo = jnp.where(m_lo, rc_vec, rc_lo)
            rc_hi = jnp.where(m_hi, rc_vec, rc_hi)
            ig_lo = jnp.where(m_lo, grow, ig_lo)
            ig_hi = jnp.where(m_hi, grow, ig_hi)
            return rc_lo, rc_hi, ig_lo, ig_hi, gi_vec

        rc_lo, rc_hi, ig_lo, ig_hi, _ = lax.fori_loop(
            0, NQ, extract,
            (zerof, zerof, zero_i, zero_i, jnp.full((16,), -1, jnp.int32)))
        rc_v[pl.ds(0, 16)] = rc_lo
        rc_v[pl.ds(16, 16)] = rc_hi
        ig_v[pl.ds(0, 16)] = ig_lo
        ig_v[pl.ds(16, 16)] = ig_hi
        pltpu.sync_copy(rc_v, rc_hbm.at[b])
        pltpu.async_copy(vid_hbm.at[ig_v], rows_v, sem).wait()
        pltpu.sync_copy(rows_v, g_hbm.at[pl.ds(b * NQ, NQ)])

    return sc_k(lg2, vid2)


def _head_body(g_ref, rc_ref, wh_ref, bh_ref, st_ref, en_ref, sc_ref):
    fq = g_ref[0]
    hh = _dot(fq, wh_ref[...]) + bh_ref[...]
    offs = jnp.tanh(hh[:, 0:2]) * 0.5
    rc0 = rc_ref[0]
    st_ref[0] = jnp.clip(rc0 - 0.05 + offs[:, 0:1], 0.0, 1.0)
    en_ref[0] = jnp.clip(rc0 + 0.05 + offs[:, 1:2], 0.0, 1.0)
    sc_ref[0] = hh[:, 2:3]


def _head(g3, rc3, w_head, b_head):
    return pl.pallas_call(
        _head_body,
        grid=(B,),
        in_specs=[
            pl.BlockSpec((1, NQ, D), lambda b: (b, 0, 0)),
            pl.BlockSpec((1, NQ, 1), lambda b: (b, 0, 0)),
            pl.BlockSpec((D, 128), lambda b: (0, 0)),
            pl.BlockSpec((1, 128), lambda b: (0, 0)),
        ],
        out_specs=[
            pl.BlockSpec((1, NQ, 1), lambda b: (b, 0, 0)),
            pl.BlockSpec((1, NQ, 1), lambda b: (b, 0, 0)),
            pl.BlockSpec((1, NQ, 1), lambda b: (b, 0, 0)),
        ],
        out_shape=[
            jax.ShapeDtypeStruct((B, NQ, 1), jnp.float32),
            jax.ShapeDtypeStruct((B, NQ, 1), jnp.float32),
            jax.ShapeDtypeStruct((B, NQ, 1), jnp.float32),
        ],
    )(g3, rc3, w_head, b_head)


def kernel(vid_feat, txt_feat, W_vid, b_vid, W_txt, b_txt, W_stage1, b_stage1,
           Wa, ba, Wb, bb, W_prop, b_prop, W_score, b_score, txt_mask):
    f32 = jnp.float32
    tm3 = txt_mask.astype(f32).reshape(B, LT, 1)
    tp = _txt_pool(txt_feat, W_txt, b_txt.reshape(1, D), tm3)

    vid = jax.nn.relu(jnp.dot(vid_feat, W_vid) + b_vid) + tp
    h = jnp.dot(vid, W_stage1) + b_stage1
    amid = jax.nn.relu(jnp.dot(h[..., 2 * D:], Wa[2]) + ba[2])

    logits = (jnp.dot(amid, Wb[2]) + bb[2])[..., 0]
    rc, g = _sc_topk_gather(logits, vid.reshape(B * T, D))

    w_head = jnp.zeros((D, 128), f32)
    w_head = w_head.at[:, 0:2].set(W_prop[2])
    w_head = w_head.at[:, 2:3].set(W_score[2])
    b_head = jnp.zeros((1, 128), f32)
    b_head = b_head.at[0, 0:2].set(b_prop[2])
    b_head = b_head.at[0, 2:3].set(b_score[2])

    st, en, sc = _head(g.reshape(B, NQ, D), rc.reshape(B, NQ, 1),
                       w_head, b_head)
    return jnp.concatenate([st, en, sc], axis=-1)


# speedup vs baseline: 1.0339x; 1.0339x over previous
import functools

import jax
import jax.numpy as jnp
from jax import lax
from jax.experimental import pallas as pl
from jax.experimental.pallas import tpu as pltpu
from jax.experimental.pallas import tpu_sc as plsc

B, T, D, LT, NQ = 32, 1024, 1024, 32, 32
TR = 512


def _dot(a, b):
    bf = jnp.bfloat16
    K = a.shape[-1]
    acc = None
    for k0 in range(0, K, 256):
        d = jnp.dot(a[:, k0:k0 + 256].astype(bf), b[k0:k0 + 256, :].astype(bf),
                    preferred_element_type=jnp.float32)
        acc = d if acc is None else acc + d
    return acc


def _txt_pool_body(txt_ref, w_ref, b_ref, tm_ref, out_ref):
    x = txt_ref[0]
    h = jnp.tanh(_dot(x, w_ref[...]) + b_ref[...]) * tm_ref[0]
    s = jnp.sum(h, axis=0, keepdims=True)
    cnt = jnp.sum(tm_ref[0])
    out_ref[0] = s / jnp.maximum(cnt, 1.0)


def _txt_pool(txt_feat, W_txt, b_txt, tm3):
    return pl.pallas_call(
        _txt_pool_body,
        grid=(B,),
        in_specs=[
            pl.BlockSpec((1, LT, D), lambda b: (b, 0, 0)),
            pl.BlockSpec((D, D), lambda b: (0, 0)),
            pl.BlockSpec((1, D), lambda b: (0, 0)),
            pl.BlockSpec((1, LT, 1), lambda b: (b, 0, 0)),
        ],
        out_specs=pl.BlockSpec((1, 1, D), lambda b: (b, 0, 0)),
        out_shape=jax.ShapeDtypeStruct((B, 1, D), jnp.float32),
    )(txt_feat, W_txt, b_txt, tm3)


def _logit_body(a_ref, wb_ref, bb_ref, out_ref):
    lg = _dot(a_ref[0], wb_ref[...])
    out_ref[0, 0, :] = lg[:, 0] + bb_ref[0, 0]


def _logit_proj(amid, wb_pad, bb2):
    return pl.pallas_call(
        _logit_body,
        grid=(B, T // TR),
        in_specs=[
            pl.BlockSpec((1, TR, D), lambda b, t: (b, t, 0)),
            pl.BlockSpec((D, 128), lambda b, t: (0, 0)),
            pl.BlockSpec((1, 1), lambda b, t: (0, 0)),
        ],
        out_specs=pl.BlockSpec((1, 1, TR), lambda b, t: (b, 0, t)),
        out_shape=jax.ShapeDtypeStruct((B, 1, T), jnp.float32),
    )(amid, wb_pad, bb2)


def _sc_topk_gather(lg2, vid2):
    info = plsc.get_sparse_core_info()
    NC = info.num_cores
    mesh = plsc.VectorSubcoreMesh(core_axis_name="c", subcore_axis_name="s")

    @functools.partial(
        pl.kernel,
        mesh=mesh,
        out_type=[jax.ShapeDtypeStruct((B, NQ), jnp.float32),
                  jax.ShapeDtypeStruct((B * NQ, D), jnp.float32)],
        scratch_types=[pltpu.VMEM((T,), jnp.float32),
                       pltpu.VMEM((NQ,), jnp.float32),
                       pltpu.VMEM((NQ,), jnp.int32),
                       pltpu.VMEM((NQ, D), jnp.float32),
                       pltpu.SemaphoreType.DMA],
    )
    def sc_k(lg_hbm, vid_hbm, rc_hbm, g_hbm, lg_v, rc_v, ig_v, rows_v, sem):
        b = lax.axis_index("s") * NC + lax.axis_index("c")
        pltpu.sync_copy(lg_hbm.at[b], lg_v)
        lanes = lax.iota(jnp.int32, 16)
        neg = jnp.full((16,), -3.0e38, jnp.float32)
        zero_i = jnp.zeros((16,), jnp.int32)

        big = jnp.full((16,), 2 ** 30, jnp.int32)
        zerof = jnp.zeros((16,), jnp.float32)
        lane0 = lanes == jnp.zeros((16,), jnp.int32)

        def _butterfly(x, op):
            for s in (8, 4, 2, 1):
                idx = jnp.bitwise_xor(lanes, jnp.full((16,), s, jnp.int32))
                x = op(x, x.at[idx].get(mode="promise_in_bounds"))
            return x

        def extract(j, carry):
            rc_lo, rc_hi, ig_lo, ig_hi, gi_prev = carry

            def scan_chunk(c, mc):
                m, mi = mc
                v = lg_v[pl.ds(c * 16, 16)]
                idxv = lanes + jnp.full((16,), c * 16, jnp.int32)
                v = jnp.where(idxv == gi_prev, neg, v)
                lg_v[pl.ds(c * 16, 16)] = v
                gt = v > m
                return (jnp.where(gt, v, m), jnp.where(gt, idxv, mi))

            m, mi = lax.fori_loop(0, T // 16, scan_chunk, (neg, zero_i))
            gm_vec = _butterfly(m, jnp.maximum)
            sel = jnp.where(m == gm_vec, mi, big)
            gi_vec = _butterfly(sel, jnp.minimum)
            rc_vec = gi_vec.astype(jnp.float32) * jnp.full(
                (16,), 1.0 / T, jnp.float32)
            ij_vec = (rc_vec * jnp.full((16,), float(T - 1), jnp.float32)
                      + jnp.full((16,), 0.5, jnp.float32)).astype(jnp.int32)
            grow = jnp.full((16,), b * T, jnp.int32) + ij_vec
            lane_lo = jnp.where(j < 16, j, 999)
            lane_hi = jnp.where(j >= 16, j - 16, 999)
            m_lo = lanes == jnp.full((16,), lane_lo, jnp.int32)
            m_hi = lanes == jnp.full((16,), lane_hi, jnp.int32)
            rc_lo = jnp.where(m_lo, rc_vec, rc_lo)
            rc_hi = jnp.where(m_hi, rc_vec, rc_hi)
            ig_lo = jnp.where(m_lo, grow, ig_lo)
            ig_hi = jnp.where(m_hi, grow, ig_hi)
            return rc_lo, rc_hi, ig_lo, ig_hi, gi_vec

        rc_lo, rc_hi, ig_lo, ig_hi, _ = lax.fori_loop(
            0, NQ, extract,
            (zerof, zerof, zero_i, zero_i, jnp.full((16,), -1, jnp.int32)))
        rc_v[pl.ds(0, 16)] = rc_lo
        rc_v[pl.ds(16, 16)] = rc_hi
        ig_v[pl.ds(0, 16)] = ig_lo
        ig_v[pl.ds(16, 16)] = ig_hi
        pltpu.sync_copy(rc_v, rc_hbm.at[b])
        pltpu.async_copy(vid_hbm.at[ig_v], rows_v, sem).wait()
        pltpu.sync_copy(rows_v, g_hbm.at[pl.ds(b * NQ, NQ)])

    return sc_k(lg2, vid2)


def _head_body(g_ref, rc_ref, wh_ref, bh_ref, st_ref, en_ref, sc_ref):
    fq = g_ref[...]
    hh = _dot(fq, wh_ref[...]) + bh_ref[...]
    offs = jnp.tanh(hh[:, 0:2]) * 0.5
    rc0 = rc_ref[...]
    st_ref[...] = jnp.clip(rc0 - 0.05 + offs[:, 0:1], 0.0, 1.0)
    en_ref[...] = jnp.clip(rc0 + 0.05 + offs[:, 1:2], 0.0, 1.0)
    sc_ref[...] = hh[:, 2:3]


def _head(g2, rc2, w_head, b_head):
    M = B * NQ
    return pl.pallas_call(
        _head_body,
        out_shape=[
            jax.ShapeDtypeStruct((M, 1), jnp.float32),
            jax.ShapeDtypeStruct((M, 1), jnp.float32),
            jax.ShapeDtypeStruct((M, 1), jnp.float32),
        ],
    )(g2, rc2, w_head, b_head)


def kernel(vid_feat, txt_feat, W_vid, b_vid, W_txt, b_txt, W_stage1, b_stage1,
           Wa, ba, Wb, bb, W_prop, b_prop, W_score, b_score, txt_mask):
    f32 = jnp.float32
    tm3 = txt_mask.astype(f32).reshape(B, LT, 1)
    tp = _txt_pool(txt_feat, W_txt, b_txt.reshape(1, D), tm3)

    vid = jax.nn.relu(jnp.dot(vid_feat, W_vid) + b_vid) + tp
    h = jnp.dot(vid, W_stage1) + b_stage1
    amid = jax.nn.relu(jnp.dot(h[..., 2 * D:], Wa[2]) + ba[2])

    logits = (jnp.dot(amid, Wb[2]) + bb[2])[..., 0]
    rc, g = _sc_topk_gather(logits, vid.reshape(B * T, D))

    w_head = jnp.zeros((D, 128), f32)
    w_head = w_head.at[:, 0:2].set(W_prop[2])
    w_head = w_head.at[:, 2:3].set(W_score[2])
    b_head = jnp.zeros((1, 128), f32)
    b_head = b_head.at[0, 0:2].set(b_prop[2])
    b_head = b_head.at[0, 2:3].set(b_score[2])

    st, en, sc = _head(g, rc.reshape(B * NQ, 1), w_head, b_head)
    out = jnp.concatenate([st, en, sc], axis=-1)
    return out.reshape(B, NQ, 3)


# speedup vs baseline: 1.0560x; 1.0214x over previous
import functools

import jax
import jax.numpy as jnp
from jax import lax
from jax.experimental import pallas as pl
from jax.experimental.pallas import tpu as pltpu
from jax.experimental.pallas import tpu_sc as plsc

B, T, D, LT, NQ = 32, 1024, 1024, 32, 32
TR = 512


def _dot(a, b):
    bf = jnp.bfloat16
    K = a.shape[-1]
    acc = None
    for k0 in range(0, K, 256):
        d = jnp.dot(a[:, k0:k0 + 256].astype(bf), b[k0:k0 + 256, :].astype(bf),
                    preferred_element_type=jnp.float32)
        acc = d if acc is None else acc + d
    return acc


def _txt_pool_body(txt_ref, w_ref, b_ref, tm_ref, out_ref):
    w = w_ref[...]
    bias = b_ref[...]
    for bb_ in range(B):
        x = txt_ref[bb_]
        h = jnp.tanh(_dot(x, w) + bias) * tm_ref[bb_]
        s = jnp.sum(h, axis=0, keepdims=True)
        cnt = jnp.sum(tm_ref[bb_])
        out_ref[bb_] = s / jnp.maximum(cnt, 1.0)


def _txt_pool(txt_feat, W_txt, b_txt, tm3):
    return pl.pallas_call(
        _txt_pool_body,
        out_shape=jax.ShapeDtypeStruct((B, 1, D), jnp.float32),
    )(txt_feat, W_txt, b_txt, tm3)


def _logit_body(a_ref, wb_ref, bb_ref, out_ref):
    lg = _dot(a_ref[0], wb_ref[...])
    out_ref[0, 0, :] = lg[:, 0] + bb_ref[0, 0]


def _logit_proj(amid, wb_pad, bb2):
    return pl.pallas_call(
        _logit_body,
        grid=(B, T // TR),
        in_specs=[
            pl.BlockSpec((1, TR, D), lambda b, t: (b, t, 0)),
            pl.BlockSpec((D, 128), lambda b, t: (0, 0)),
            pl.BlockSpec((1, 1), lambda b, t: (0, 0)),
        ],
        out_specs=pl.BlockSpec((1, 1, TR), lambda b, t: (b, 0, t)),
        out_shape=jax.ShapeDtypeStruct((B, 1, T), jnp.float32),
    )(amid, wb_pad, bb2)


def _sc_topk_gather(lg2, vid2):
    info = plsc.get_sparse_core_info()
    NC = info.num_cores
    mesh = plsc.VectorSubcoreMesh(core_axis_name="c", subcore_axis_name="s")

    @functools.partial(
        pl.kernel,
        mesh=mesh,
        out_type=[jax.ShapeDtypeStruct((B, NQ), jnp.float32),
                  jax.ShapeDtypeStruct((B * NQ, D), jnp.float32)],
        scratch_types=[pltpu.VMEM((T,), jnp.float32),
                       pltpu.VMEM((NQ,), jnp.float32),
                       pltpu.VMEM((NQ,), jnp.int32),
                       pltpu.VMEM((NQ, D), jnp.float32),
                       pltpu.SemaphoreType.DMA],
    )
    def sc_k(lg_hbm, vid_hbm, rc_hbm, g_hbm, lg_v, rc_v, ig_v, rows_v, sem):
        b = lax.axis_index("s") * NC + lax.axis_index("c")
        pltpu.sync_copy(lg_hbm.at[b], lg_v)
        lanes = lax.iota(jnp.int32, 16)
        neg = jnp.full((16,), -3.0e38, jnp.float32)
        zero_i = jnp.zeros((16,), jnp.int32)

        big = jnp.full((16,), 2 ** 30, jnp.int32)
        zerof = jnp.zeros((16,), jnp.float32)
        lane0 = lanes == jnp.zeros((16,), jnp.int32)

        def _butterfly(x, op):
            for s in (8, 4, 2, 1):
                idx = jnp.bitwise_xor(lanes, jnp.full((16,), s, jnp.int32))
                x = op(x, x.at[idx].get(mode="promise_in_bounds"))
            return x

        def extract(j, carry):
            rc_lo, rc_hi, ig_lo, ig_hi, gi_prev = carry

            def scan_chunk(c, mc):
                m, mi = mc
                v = lg_v[pl.ds(c * 16, 16)]
                idxv = lanes + jnp.full((16,), c * 16, jnp.int32)
                v = jnp.where(idxv == gi_prev, neg, v)
                lg_v[pl.ds(c * 16, 16)] = v
                gt = v > m
                return (jnp.where(gt, v, m), jnp.where(gt, idxv, mi))

            m, mi = lax.fori_loop(0, T // 16, scan_chunk, (neg, zero_i))
            gm_vec = _butterfly(m, jnp.maximum)
            sel = jnp.where(m == gm_vec, mi, big)
            gi_vec = _butterfly(sel, jnp.minimum)
            rc_vec = gi_vec.astype(jnp.float32) * jnp.full(
                (16,), 1.0 / T, jnp.float32)
            ij_vec = (rc_vec * jnp.full((16,), float(T - 1), jnp.float32)
                      + jnp.full((16,), 0.5, jnp.float32)).astype(jnp.int32)
            grow = jnp.full((16,), b * T, jnp.int32) + ij_vec
            lane_lo = jnp.where(j < 16, j, 999)
            lane_hi = jnp.where(j >= 16, j - 16, 999)
            m_lo = lanes == jnp.full((16,), lane_lo, jnp.int32)
            m_hi = lanes == jnp.full((16,), lane_hi, jnp.int32)
            rc_lo = jnp.where(m_lo, rc_vec, rc_lo)
            rc_hi = jnp.where(m_hi, rc_vec, rc_hi)
            ig_lo = jnp.where(m_lo, grow, ig_lo)
            ig_hi = jnp.where(m_hi, grow, ig_hi)
            return rc_lo, rc_hi, ig_lo, ig_hi, gi_vec

        rc_lo, rc_hi, ig_lo, ig_hi, _ = lax.fori_loop(
            0, NQ, extract,
            (zerof, zerof, zero_i, zero_i, jnp.full((16,), -1, jnp.int32)))
        rc_v[pl.ds(0, 16)] = rc_lo
        rc_v[pl.ds(16, 16)] = rc_hi
        ig_v[pl.ds(0, 16)] = ig_lo
        ig_v[pl.ds(16, 16)] = ig_hi
        pltpu.sync_copy(rc_v, rc_hbm.at[b])
        pltpu.async_copy(vid_hbm.at[ig_v], rows_v, sem).wait()
        pltpu.sync_copy(rows_v, g_hbm.at[pl.ds(b * NQ, NQ)])

    return sc_k(lg2, vid2)


def _head_body(g_ref, rc_ref, wh_ref, bh_ref, st_ref, en_ref, sc_ref):
    fq = g_ref[...]
    hh = _dot(fq, wh_ref[...]) + bh_ref[...]
    offs = jnp.tanh(hh[:, 0:2]) * 0.5
    rc0 = rc_ref[...]
    st_ref[...] = jnp.clip(rc0 - 0.05 + offs[:, 0:1], 0.0, 1.0)
    en_ref[...] = jnp.clip(rc0 + 0.05 + offs[:, 1:2], 0.0, 1.0)
    sc_ref[...] = hh[:, 2:3]


def _head(g2, rc2, w_head, b_head):
    M = B * NQ
    return pl.pallas_call(
        _head_body,
        out_shape=[
            jax.ShapeDtypeStruct((M, 1), jnp.float32),
            jax.ShapeDtypeStruct((M, 1), jnp.float32),
            jax.ShapeDtypeStruct((M, 1), jnp.float32),
        ],
    )(g2, rc2, w_head, b_head)


def kernel(vid_feat, txt_feat, W_vid, b_vid, W_txt, b_txt, W_stage1, b_stage1,
           Wa, ba, Wb, bb, W_prop, b_prop, W_score, b_score, txt_mask):
    f32 = jnp.float32
    tm3 = txt_mask.astype(f32).reshape(B, LT, 1)
    tp = _txt_pool(txt_feat, W_txt, b_txt.reshape(1, D), tm3)

    vid = jax.nn.relu(jnp.dot(vid_feat, W_vid) + b_vid) + tp
    h = jnp.dot(vid, W_stage1) + b_stage1
    amid = jax.nn.relu(jnp.dot(h[..., 2 * D:], Wa[2]) + ba[2])

    logits = (jnp.dot(amid, Wb[2]) + bb[2])[..., 0]
    rc, g = _sc_topk_gather(logits, vid.reshape(B * T, D))

    w_head = jnp.zeros((D, 128), f32)
    w_head = w_head.at[:, 0:2].set(W_prop[2])
    w_head = w_head.at[:, 2:3].set(W_score[2])
    b_head = jnp.zeros((1, 128), f32)
    b_head = b_head.at[0, 0:2].set(b_prop[2])
    b_head = b_head.at[0, 2:3].set(b_score[2])

    st, en, sc = _head(g, rc.reshape(B * NQ, 1), w_head, b_head)
    out = jnp.concatenate([st, en, sc], axis=-1)
    return out.reshape(B, NQ, 3)
